# Initial kernel scaffold; baseline (speedup 1.0000x reference)
import jax
import jax.numpy as jnp
from jax.experimental import pallas as pl


def kernel(input, embedding_weight):
    raise NotImplementedError("write your pallas kernel here")


# speedup vs baseline: 6.9978x; 6.9978x over previous
import functools

import jax
import jax.numpy as jnp
from jax import lax
from jax.experimental import pallas as pl
from jax.experimental.pallas import tpu as pltpu
from jax.experimental.pallas import tpu_sc as plsc

K = 8192
D = 256
N = 8192
TN = 512
TK = 1024

NC, NS = 2, 16
NW = NC * NS
BPW = N // NW
CH = 128


def _argmin_body(x_ref, w_ref, idx_ref, min_ref):
    j = pl.program_id(1)
    x = x_ref[...]
    w = w_ref[...]
    mm = lax.dot_general(x, w, (((1,), (1,)), ((), ())),
                         preferred_element_type=jnp.float32)
    xn = jnp.sum(x * x, axis=1, keepdims=True)
    ones = jnp.ones((1, D), jnp.float32)
    wn = lax.dot_general(ones, w * w, (((1,), (1,)), ((), ())),
                         preferred_element_type=jnp.float32)
    dist = (xn + wn) - 2.0 * mm
    local_min = jnp.min(dist, axis=1, keepdims=True)
    ids = lax.broadcasted_iota(jnp.int32, (TN, TK), 1)
    masked = jnp.where(dist == local_min, ids, K)
    local_arg = jnp.min(masked, axis=1, keepdims=True) + j * TK

    @pl.when(j == 0)
    def _():
        min_ref[...] = local_min
        idx_ref[...] = local_arg

    @pl.when(j > 0)
    def _():
        better = local_min < min_ref[...]
        min_ref[...] = jnp.where(better, local_min, min_ref[...])
        idx_ref[...] = jnp.where(better, local_arg, idx_ref[...])


_argmin_call = pl.pallas_call(
    _argmin_body,
    grid=(N // TN, K // TK),
    in_specs=[
        pl.BlockSpec((TN, D), lambda i, j: (i, 0)),
        pl.BlockSpec((TK, D), lambda i, j: (j, 0)),
    ],
    out_specs=pl.BlockSpec((TN, 1), lambda i, j: (i, 0)),
    out_shape=jax.ShapeDtypeStruct((N, 1), jnp.int32),
    scratch_shapes=[pltpu.VMEM((TN, 1), jnp.float32)],
    compiler_params=pltpu.CompilerParams(
        dimension_semantics=("parallel", "arbitrary")),
)


def _gather_body(table_hbm, idx_hbm, out_hbm, idx_v, rows_v, sem):
    wid = lax.axis_index("s") * NC + lax.axis_index("c")
    base = wid * BPW
    pltpu.sync_copy(idx_hbm.at[pl.ds(wid * (BPW // CH), BPW // CH)], idx_v)
    copies = []
    for c in range(BPW // CH):
        copies.append(pltpu.async_copy(
            table_hbm.at[idx_v.at[c]], rows_v.at[pl.ds(c * CH, CH)], sem))
    for cp in copies:
        cp.wait()
    pltpu.sync_copy(rows_v, out_hbm.at[pl.ds(base, BPW)])


@functools.cache
def _gather_call():
    return pl.kernel(
        _gather_body,
        out_type=jax.ShapeDtypeStruct((N, D), jnp.float32),
        mesh=plsc.VectorSubcoreMesh(core_axis_name="c", subcore_axis_name="s",
                                    num_cores=NC, num_subcores=NS),
        scratch_types=[
            pltpu.VMEM((BPW // CH, CH), jnp.int32),
            pltpu.VMEM((BPW, D), jnp.float32),
            pltpu.SemaphoreType.DMA,
        ],
    )


def kernel(input, embedding_weight):
    x = jnp.transpose(input, (0, 2, 3, 1)).reshape(N, D)
    idx = _argmin_call(x, embedding_weight)
    idx_grid = idx.reshape(N // CH, CH)
    rows = _gather_call()(embedding_weight, idx_grid)
    out = rows.reshape(input.shape[0], 32, 32, D)
    return jnp.transpose(out, (0, 3, 1, 2))


# speedup vs baseline: 7.0021x; 1.0006x over previous
import functools

import jax
import jax.numpy as jnp
from jax import lax
from jax.experimental import pallas as pl
from jax.experimental.pallas import tpu as pltpu
from jax.experimental.pallas import tpu_sc as plsc

K = 8192
D = 256
N = 8192
TN = 512
TK = 1024

NC, NS = 2, 16
NW = NC * NS
BPW = N // NW
CH = 128


def _argmin_body(x_ref, w_ref, idx_ref, min_s, arg_s):
    j = pl.program_id(0)
    i = pl.program_id(1)
    x = x_ref[...]
    w = w_ref[...]
    mm2 = lax.dot_general(x * -2.0, w, (((1,), (1,)), ((), ())),
                          preferred_element_type=jnp.float32)
    xn = jnp.sum(x * x, axis=1, keepdims=True)
    ones = jnp.ones((1, D), jnp.float32)
    wn = lax.dot_general(ones, w * w, (((1,), (1,)), ((), ())),
                         preferred_element_type=jnp.float32)
    dist = (xn + wn) + mm2
    local_min = jnp.min(dist, axis=1, keepdims=True)
    ids = lax.broadcasted_iota(jnp.int32, (TN, TK), 1)
    masked = jnp.where(dist == local_min, ids, K)
    local_arg = jnp.min(masked, axis=1, keepdims=True) + j * TK

    slab = pl.ds(i * TN, TN)

    @pl.when(j == 0)
    def _():
        min_s[slab, :] = local_min
        arg_s[slab, :] = local_arg

    @pl.when(j > 0)
    def _():
        better = local_min < min_s[slab, :]
        min_s[slab, :] = jnp.where(better, local_min, min_s[slab, :])
        arg_s[slab, :] = jnp.where(better, local_arg, arg_s[slab, :])

    @pl.when(j == K // TK - 1)
    def _():
        idx_ref[...] = arg_s[slab, :]


_argmin_call = pl.pallas_call(
    _argmin_body,
    grid=(K // TK, N // TN),
    in_specs=[
        pl.BlockSpec((TN, D), lambda j, i: (i, 0)),
        pl.BlockSpec((TK, D), lambda j, i: (j, 0)),
    ],
    out_specs=pl.BlockSpec((TN, 1), lambda j, i: (i, 0)),
    out_shape=jax.ShapeDtypeStruct((N, 1), jnp.int32),
    scratch_shapes=[pltpu.VMEM((N, 1), jnp.float32),
                    pltpu.VMEM((N, 1), jnp.int32)],
    compiler_params=pltpu.CompilerParams(
        dimension_semantics=("arbitrary", "arbitrary")),
)


def _gather_body(table_hbm, idx_hbm, out_hbm, idx_v, rows_v, sem):
    wid = lax.axis_index("s") * NC + lax.axis_index("c")
    base = wid * BPW
    pltpu.sync_copy(idx_hbm.at[pl.ds(wid * (BPW // CH), BPW // CH)], idx_v)
    copies = []
    for c in range(BPW // CH):
        copies.append(pltpu.async_copy(
            table_hbm.at[idx_v.at[c]], rows_v.at[pl.ds(c * CH, CH)], sem))
    for cp in copies:
        cp.wait()
    pltpu.sync_copy(rows_v, out_hbm.at[pl.ds(base, BPW)])


@functools.cache
def _gather_call():
    return pl.kernel(
        _gather_body,
        out_type=jax.ShapeDtypeStruct((N, D), jnp.float32),
        mesh=plsc.VectorSubcoreMesh(core_axis_name="c", subcore_axis_name="s",
                                    num_cores=NC, num_subcores=NS),
        scratch_types=[
            pltpu.VMEM((BPW // CH, CH), jnp.int32),
            pltpu.VMEM((BPW, D), jnp.float32),
            pltpu.SemaphoreType.DMA,
        ],
    )


def kernel(input, embedding_weight):
    x = jnp.transpose(input, (0, 2, 3, 1)).reshape(N, D)
    idx = _argmin_call(x, embedding_weight)
    idx_grid = idx.reshape(N // CH, CH)
    rows = _gather_call()(embedding_weight, idx_grid)
    out = rows.reshape(input.shape[0], 32, 32, D)
    return jnp.transpose(out, (0, 3, 1, 2))
